# baseline (speedup 1.0000x reference)
import functools

import jax
import jax.numpy as jnp
from jax import lax
from jax.experimental import pallas as pl
from jax.experimental.pallas import tpu as pltpu
from jax.experimental.pallas import tpu_sc as plsc

N = 10000
E = 160000
D = 256
HD = 128
NP = 10240
DISCARD = 10016
NT = 16
CH = 64
NCH = 158
EPT = NCH * CH
EP = NT * EPT
RPT = NP // NT

_mesh = plsc.VectorSubcoreMesh(core_axis_name="c", subcore_axis_name="s")


def _zero_1d(ref, n):
    z = jnp.zeros((16,), jnp.float32)

    def body(k, _):
        ref[pl.ds(k * 16, 16)] = z
        return 0

    lax.fori_loop(0, n // 16, body, 0)


def _zero_2d(ref, rows):
    z = jnp.zeros((16,), jnp.float32)

    def body(q, _):
        i = q // 8
        k = q - i * 8
        ref[i, pl.ds(k * 16, 16)] = z
        return 0

    lax.fori_loop(0, rows * 8, body, 0)


@functools.partial(
    pl.kernel,
    out_type=[
        jax.ShapeDtypeStruct((NP, HD), jnp.float32),
        jax.ShapeDtypeStruct((NP, HD), jnp.float32),
        jax.ShapeDtypeStruct((NP,), jnp.float32),
    ],
    mesh=_mesh,
    scratch_types=[
        pltpu.VMEM((EPT,), jnp.int32),
        pltpu.VMEM((EPT,), jnp.int32),
        pltpu.VMEM((CH, HD), jnp.float32),
        pltpu.VMEM((CH, HD), jnp.float32),
        pltpu.VMEM((CH, HD), jnp.float32),
        pltpu.VMEM((512,), jnp.float32),
        pltpu.VMEM((RPT,), jnp.float32),
        pltpu.VMEM_SHARED((NP, HD), jnp.float32),
        pltpu.VMEM_SHARED((NP,), jnp.float32),
        pltpu.SemaphoreType.DMA,
        pltpu.SemaphoreType.DMA,
        pltpu.SemaphoreType.DMA,
        pltpu.SemaphoreType.DMA,
        pltpu.SemaphoreType.DMA,
        pltpu.SemaphoreType.DMA,
    ],
    compiler_params=pltpu.CompilerParams(needs_layout_passes=False),
)
def _sc_pass1(x0_hbm, x1_hbm, src_hbm, dst_hbm, agg0_hbm, agg1_hbm, cnt_hbm,
              sidx, didx, rows0, rows1, rows2, onesb, zbuf, agg_sh, cnt_sh,
              sem0, sem1, sem2, ssem0, ssem1, ssem2):
    c = lax.axis_index("c")
    s = lax.axis_index("s")
    ones = jnp.ones((16,), jnp.float32)
    rbufs = (rows0, rows1, rows2)
    sems = (sem0, sem1, sem2)
    ssems = (ssem0, ssem1, ssem2)

    _zero_2d(rows0, CH)
    for b in range(RPT // CH):
        pltpu.sync_copy(rows0, agg_sh.at[pl.ds(s * RPT + b * CH, CH)])
    _zero_1d(zbuf, RPT)
    pltpu.sync_copy(zbuf, cnt_sh.at[pl.ds(s * RPT, RPT)])

    def fill_ones(k, _):
        onesb[pl.ds(k * 16, 16)] = ones
        return 0

    lax.fori_loop(0, 512 // 16, fill_ones, 0)
    pltpu.sync_copy(src_hbm.at[pl.ds(s * EPT, EPT)], sidx)
    pltpu.sync_copy(dst_hbm.at[pl.ds(s * EPT, EPT)], didx)
    plsc.subcore_barrier()

    def edge_loop(x_hbm):
        for b in range(2):
            pltpu.async_copy(
                x_hbm.at[sidx.at[pl.ds(b * CH, CH)]], rbufs[b], sems[b])

        def step(j, bc, bp):
            pltpu.make_async_copy(
                x_hbm.at[sidx.at[pl.ds(j * CH, CH)]], rbufs[bc],
                sems[bc]).wait()
            pltpu.async_copy(rbufs[bc],
                             agg_sh.at[didx.at[pl.ds(j * CH, CH)]],
                             ssems[bc], add=True)

            @pl.when(j >= 1)
            def _():
                pltpu.make_async_copy(
                    rbufs[bp],
                    agg_sh.at[didx.at[pl.ds((j - 1) * CH, CH)]],
                    ssems[bp]).wait()

            @pl.when(j + 2 < NCH)
            def _():
                pltpu.async_copy(
                    x_hbm.at[sidx.at[pl.ds((j + 2) * CH, CH)]], rbufs[bp],
                    sems[bp])

        def body(i, _):
            j = 3 * i
            step(j, 0, 2)
            step(j + 1, 1, 0)
            step(j + 2, 2, 1)
            return 0

        lax.fori_loop(0, NCH // 3, body, 0)
        step(NCH - 2, 0, 2)
        step(NCH - 1, 1, 0)
        pltpu.make_async_copy(
            rbufs[1], agg_sh.at[didx.at[pl.ds((NCH - 1) * CH, CH)]],
            ssems[1]).wait()

    @pl.when(c == 0)
    def _():
        edge_loop(x0_hbm)
        for b in range(EPT // 512):
            pltpu.sync_copy(
                onesb, cnt_sh.at[didx.at[pl.ds(b * 512, 512)]], add=True)
        rem = EPT - (EPT // 512) * 512
        if rem:
            pltpu.sync_copy(
                onesb.at[pl.ds(0, rem)],
                cnt_sh.at[didx.at[pl.ds(EPT - rem, rem)]], add=True)

    @pl.when(c == 1)
    def _():
        edge_loop(x1_hbm)

    plsc.subcore_barrier()

    @pl.when(c == 0)
    def _():
        pltpu.sync_copy(agg_sh.at[pl.ds(s * RPT, RPT)],
                        agg0_hbm.at[pl.ds(s * RPT, RPT)])
        pltpu.sync_copy(cnt_sh.at[pl.ds(s * RPT, RPT)],
                        cnt_hbm.at[pl.ds(s * RPT, RPT)])

    @pl.when(c == 1)
    def _():
        pltpu.sync_copy(agg_sh.at[pl.ds(s * RPT, RPT)],
                        agg1_hbm.at[pl.ds(s * RPT, RPT)])


_BLK = 512


def _tc_body(cnt_ref, x_ref, a0_ref, a1_ref, w1l_ref, b1_ref, w1r_ref,
             w2_ref, tu_ref):
    dn = (((1,), (1,)), ((), ()))
    r = 1.0 / jnp.maximum(cnt_ref[...], 1.0)
    m0 = a0_ref[...] * r
    m1 = a1_ref[...] * r
    w1l = w1l_ref[...]
    acc = lax.dot_general(m0, w1l[:, :HD], dn,
                          preferred_element_type=jnp.float32)
    acc = acc + lax.dot_general(m1, w1l[:, HD:], dn,
                                preferred_element_type=jnp.float32)
    acc = acc + lax.dot_general(x_ref[...], w1r_ref[...], dn,
                                preferred_element_type=jnp.float32)
    h = jnp.maximum(acc + b1_ref[...], 0.0)
    tu_ref[...] = lax.dot_general(h, w2_ref[...], dn,
                                  preferred_element_type=jnp.float32)


def _tc_dense(cnt, x, agg0, agg1, W1_l, b1_l, W1_r, W2):
    grid = (NP // _BLK,)
    return pl.pallas_call(
        _tc_body,
        grid=grid,
        in_specs=[
            pl.BlockSpec((_BLK, 1), lambda i: (i, 0)),
            pl.BlockSpec((_BLK, D), lambda i: (i, 0)),
            pl.BlockSpec((_BLK, HD), lambda i: (i, 0)),
            pl.BlockSpec((_BLK, HD), lambda i: (i, 0)),
            pl.BlockSpec((D, D), lambda i: (0, 0)),
            pl.BlockSpec((1, D), lambda i: (0, 0)),
            pl.BlockSpec((D, D), lambda i: (0, 0)),
            pl.BlockSpec((2, D), lambda i: (0, 0)),
        ],
        out_specs=pl.BlockSpec((_BLK, 2), lambda i: (i, 0)),
        out_shape=jax.ShapeDtypeStruct((NP, 2), jnp.float32),
    )(cnt, x, agg0, agg1, W1_l, b1_l, W1_r, W2)


_V2 = EPT // 16


@functools.partial(
    pl.kernel,
    out_type=jax.ShapeDtypeStruct((NP,), jnp.float32),
    mesh=_mesh,
    scratch_types=[
        pltpu.VMEM((NP,), jnp.float32),
        pltpu.VMEM((NP,), jnp.float32),
        pltpu.VMEM((EPT,), jnp.int32),
        pltpu.VMEM((EPT,), jnp.int32),
        pltpu.VMEM((NT, RPT), jnp.float32),
        pltpu.VMEM((RPT,), jnp.float32),
        pltpu.VMEM((RPT,), jnp.float32),
        pltpu.VMEM((16,), jnp.float32),
        pltpu.VMEM((RPT,), jnp.float32),
        pltpu.VMEM_SHARED((NT, NP), jnp.float32),
    ],
    compiler_params=pltpu.CompilerParams(needs_layout_passes=False),
)
def _sc_pass2(t_hbm, u_hbm, cnt_hbm, src_hbm, dst_hbm, b2_hbm, out_hbm,
              tl, sl, sidxl, didxl, cslice, cntv, uv, b2v, cout, parts_sh):
    c = lax.axis_index("c")
    s = lax.axis_index("s")

    @pl.when(c == 0)
    def _():
        pltpu.sync_copy(t_hbm, tl)
        pltpu.sync_copy(src_hbm.at[pl.ds(s * EPT, EPT)], sidxl)
        pltpu.sync_copy(dst_hbm.at[pl.ds(s * EPT, EPT)], didxl)
        _zero_1d(sl, NP)

        def body(j, _):
            sv = sidxl[pl.ds(j * 16, 16)]
            dv = didxl[pl.ds(j * 16, 16)]
            vals = plsc.load_gather(tl, [sv])
            plsc.addupdate_scatter(sl, [dv], vals)
            return 0

        lax.fori_loop(0, _V2, body, 0)
        pltpu.sync_copy(sl, parts_sh.at[s])
        plsc.subcore_barrier()

        for r in range(NT):
            pltpu.sync_copy(parts_sh.at[r, pl.ds(s * RPT, RPT)], cslice.at[r])
        pltpu.sync_copy(cnt_hbm.at[pl.ds(s * RPT, RPT)], cntv)
        pltpu.sync_copy(u_hbm.at[pl.ds(s * RPT, RPT)], uv)
        pltpu.sync_copy(b2_hbm, b2v)
        b2 = b2v[...]

        def comb(k, _):
            v = cslice[0, pl.ds(k * 16, 16)]
            for r in range(1, NT):
                v = v + cslice[r, pl.ds(k * 16, 16)]
            v = v / jnp.maximum(cntv[pl.ds(k * 16, 16)], 1.0)
            cout[pl.ds(k * 16, 16)] = v + b2 + uv[pl.ds(k * 16, 16)]
            return 0

        lax.fori_loop(0, RPT // 16, comb, 0)
        pltpu.sync_copy(cout, out_hbm.at[pl.ds(s * RPT, RPT)])


def kernel(x, edge_index, W1_l, b1_l, W1_r, W2_l, b2_l, W2_r):
    src = jnp.concatenate(
        [edge_index[0], jnp.zeros((EP - E,), jnp.int32)])
    dst = jnp.concatenate(
        [edge_index[1], jnp.full((EP - E,), DISCARD, jnp.int32)])
    x0 = x[:, :HD]
    x1 = x[:, HD:]
    agg0, agg1, cnt = _sc_pass1(x0, x1, src, dst)

    W2 = jnp.concatenate([W2_l, W2_r], axis=0)
    tu = _tc_dense(cnt.reshape(NP, 1), x, agg0, agg1, W1_l,
                   b1_l.reshape(1, D), W1_r, W2)
    t = tu[:, 0]
    u = tu[:, 1]

    b2b = jnp.broadcast_to(b2_l, (16,))
    out = _sc_pass2(t, u, cnt, src, dst, b2b)
    return out[:N]


# speedup vs baseline: 1.0237x; 1.0237x over previous
import functools

import jax
import jax.numpy as jnp
from jax import lax
from jax.experimental import pallas as pl
from jax.experimental.pallas import tpu as pltpu
from jax.experimental.pallas import tpu_sc as plsc

N = 10000
E = 160000
D = 256
HD = 128
NP = 10240
DISCARD = 10016
NT = 16
CH = 32
NCH = 316
NB = 6
EPT = NCH * CH
EP = NT * EPT
RPT = NP // NT

_mesh = plsc.VectorSubcoreMesh(core_axis_name="c", subcore_axis_name="s")


def _zero_1d(ref, n):
    z = jnp.zeros((16,), jnp.float32)

    def body(k, _):
        ref[pl.ds(k * 16, 16)] = z
        return 0

    lax.fori_loop(0, n // 16, body, 0)


def _zero_2d(ref, rows):
    z = jnp.zeros((16,), jnp.float32)

    def body(q, _):
        i = q // 8
        k = q - i * 8
        ref[i, pl.ds(k * 16, 16)] = z
        return 0

    lax.fori_loop(0, rows * 8, body, 0)


@functools.partial(
    pl.kernel,
    out_type=[
        jax.ShapeDtypeStruct((NP, HD), jnp.float32),
        jax.ShapeDtypeStruct((NP, HD), jnp.float32),
        jax.ShapeDtypeStruct((NP,), jnp.float32),
    ],
    mesh=_mesh,
    scratch_types=[
        pltpu.VMEM((EPT,), jnp.int32),
        pltpu.VMEM((EPT,), jnp.int32),
    ] + [pltpu.VMEM((CH, HD), jnp.float32)] * NB + [
        pltpu.VMEM((512,), jnp.float32),
        pltpu.VMEM((RPT,), jnp.float32),
        pltpu.VMEM_SHARED((NP, HD), jnp.float32),
        pltpu.VMEM_SHARED((NP,), jnp.float32),
    ] + [pltpu.SemaphoreType.DMA] * NB,
    compiler_params=pltpu.CompilerParams(needs_layout_passes=False),
)
def _sc_pass1(x0_hbm, x1_hbm, src_hbm, dst_hbm, agg0_hbm, agg1_hbm, cnt_hbm,
              sidx, didx, *rest):
    rbufs = rest[:NB]
    onesb, zbuf, agg_sh, cnt_sh = rest[NB:NB + 4]
    sems = rest[NB + 4:]
    c = lax.axis_index("c")
    s = lax.axis_index("s")
    ones = jnp.ones((16,), jnp.float32)
    rows0 = rbufs[0]

    _zero_2d(rows0, CH)
    for b in range(RPT // CH):
        pltpu.sync_copy(rows0, agg_sh.at[pl.ds(s * RPT + b * CH, CH)])
    _zero_1d(zbuf, RPT)
    pltpu.sync_copy(zbuf, cnt_sh.at[pl.ds(s * RPT, RPT)])

    def fill_ones(k, _):
        onesb[pl.ds(k * 16, 16)] = ones
        return 0

    lax.fori_loop(0, 512 // 16, fill_ones, 0)
    pltpu.sync_copy(src_hbm.at[pl.ds(s * EPT, EPT)], sidx)
    pltpu.sync_copy(dst_hbm.at[pl.ds(s * EPT, EPT)], didx)
    plsc.subcore_barrier()

    def edge_loop(x_hbm):
        for b in range(NB):
            pltpu.async_copy(
                x_hbm.at[sidx.at[pl.ds(b * CH, CH)]], rbufs[b], sems[b])

        def step(j, b):
            pltpu.make_async_copy(
                x_hbm.at[sidx.at[pl.ds(j * CH, CH)]], rbufs[b],
                sems[b]).wait()
            pltpu.sync_copy(rbufs[b], agg_sh.at[didx.at[pl.ds(j * CH, CH)]],
                            add=True)

            @pl.when(j + NB < NCH)
            def _():
                pltpu.async_copy(
                    x_hbm.at[sidx.at[pl.ds((j + NB) * CH, CH)]], rbufs[b],
                    sems[b])

        def body(i, _):
            j = NB * i
            for b in range(NB):
                step(j + b, b)
            return 0

        lax.fori_loop(0, NCH // NB, body, 0)
        for b in range(NCH - (NCH // NB) * NB):
            step((NCH // NB) * NB + b, b)

    @pl.when(c == 0)
    def _():
        edge_loop(x0_hbm)
        for b in range(EPT // 512):
            pltpu.sync_copy(
                onesb, cnt_sh.at[didx.at[pl.ds(b * 512, 512)]], add=True)
        rem = EPT - (EPT // 512) * 512
        if rem:
            pltpu.sync_copy(
                onesb.at[pl.ds(0, rem)],
                cnt_sh.at[didx.at[pl.ds(EPT - rem, rem)]], add=True)

    @pl.when(c == 1)
    def _():
        edge_loop(x1_hbm)

    plsc.subcore_barrier()

    @pl.when(c == 0)
    def _():
        pltpu.sync_copy(agg_sh.at[pl.ds(s * RPT, RPT)],
                        agg0_hbm.at[pl.ds(s * RPT, RPT)])
        pltpu.sync_copy(cnt_sh.at[pl.ds(s * RPT, RPT)],
                        cnt_hbm.at[pl.ds(s * RPT, RPT)])

    @pl.when(c == 1)
    def _():
        pltpu.sync_copy(agg_sh.at[pl.ds(s * RPT, RPT)],
                        agg1_hbm.at[pl.ds(s * RPT, RPT)])


_BLK = 512


def _tc_body(cnt_ref, x_ref, a0_ref, a1_ref, w1l_ref, b1_ref, w1r_ref,
             w2_ref, tu_ref):
    dn = (((1,), (1,)), ((), ()))
    r = 1.0 / jnp.maximum(cnt_ref[...], 1.0)
    m0 = a0_ref[...] * r
    m1 = a1_ref[...] * r
    w1l = w1l_ref[...]
    acc = lax.dot_general(m0, w1l[:, :HD], dn,
                          preferred_element_type=jnp.float32)
    acc = acc + lax.dot_general(m1, w1l[:, HD:], dn,
                                preferred_element_type=jnp.float32)
    acc = acc + lax.dot_general(x_ref[...], w1r_ref[...], dn,
                                preferred_element_type=jnp.float32)
    h = jnp.maximum(acc + b1_ref[...], 0.0)
    tu_ref[...] = lax.dot_general(h, w2_ref[...], dn,
                                  preferred_element_type=jnp.float32)


def _tc_dense(cnt, x, agg0, agg1, W1_l, b1_l, W1_r, W2):
    grid = (NP // _BLK,)
    return pl.pallas_call(
        _tc_body,
        grid=grid,
        in_specs=[
            pl.BlockSpec((_BLK, 1), lambda i: (i, 0)),
            pl.BlockSpec((_BLK, D), lambda i: (i, 0)),
            pl.BlockSpec((_BLK, HD), lambda i: (i, 0)),
            pl.BlockSpec((_BLK, HD), lambda i: (i, 0)),
            pl.BlockSpec((D, D), lambda i: (0, 0)),
            pl.BlockSpec((1, D), lambda i: (0, 0)),
            pl.BlockSpec((D, D), lambda i: (0, 0)),
            pl.BlockSpec((2, D), lambda i: (0, 0)),
        ],
        out_specs=pl.BlockSpec((_BLK, 2), lambda i: (i, 0)),
        out_shape=jax.ShapeDtypeStruct((NP, 2), jnp.float32),
    )(cnt, x, agg0, agg1, W1_l, b1_l, W1_r, W2)


_V2 = EPT // 16


@functools.partial(
    pl.kernel,
    out_type=jax.ShapeDtypeStruct((NP,), jnp.float32),
    mesh=_mesh,
    scratch_types=[
        pltpu.VMEM((NP,), jnp.float32),
        pltpu.VMEM((NP,), jnp.float32),
        pltpu.VMEM((EPT,), jnp.int32),
        pltpu.VMEM((EPT,), jnp.int32),
        pltpu.VMEM((NT, RPT), jnp.float32),
        pltpu.VMEM((RPT,), jnp.float32),
        pltpu.VMEM((RPT,), jnp.float32),
        pltpu.VMEM((16,), jnp.float32),
        pltpu.VMEM((RPT,), jnp.float32),
        pltpu.VMEM_SHARED((NT, NP), jnp.float32),
    ],
    compiler_params=pltpu.CompilerParams(needs_layout_passes=False),
)
def _sc_pass2(t_hbm, u_hbm, cnt_hbm, src_hbm, dst_hbm, b2_hbm, out_hbm,
              tl, sl, sidxl, didxl, cslice, cntv, uv, b2v, cout, parts_sh):
    c = lax.axis_index("c")
    s = lax.axis_index("s")

    @pl.when(c == 0)
    def _():
        pltpu.sync_copy(t_hbm, tl)
        pltpu.sync_copy(src_hbm.at[pl.ds(s * EPT, EPT)], sidxl)
        pltpu.sync_copy(dst_hbm.at[pl.ds(s * EPT, EPT)], didxl)
        _zero_1d(sl, NP)

        def body(j, _):
            sv = sidxl[pl.ds(j * 16, 16)]
            dv = didxl[pl.ds(j * 16, 16)]
            vals = plsc.load_gather(tl, [sv])
            plsc.addupdate_scatter(sl, [dv], vals)
            return 0

        lax.fori_loop(0, _V2, body, 0)
        pltpu.sync_copy(sl, parts_sh.at[s])
        plsc.subcore_barrier()

        for r in range(NT):
            pltpu.sync_copy(parts_sh.at[r, pl.ds(s * RPT, RPT)], cslice.at[r])
        pltpu.sync_copy(cnt_hbm.at[pl.ds(s * RPT, RPT)], cntv)
        pltpu.sync_copy(u_hbm.at[pl.ds(s * RPT, RPT)], uv)
        pltpu.sync_copy(b2_hbm, b2v)
        b2 = b2v[...]

        def comb(k, _):
            v = cslice[0, pl.ds(k * 16, 16)]
            for r in range(1, NT):
                v = v + cslice[r, pl.ds(k * 16, 16)]
            v = v / jnp.maximum(cntv[pl.ds(k * 16, 16)], 1.0)
            cout[pl.ds(k * 16, 16)] = v + b2 + uv[pl.ds(k * 16, 16)]
            return 0

        lax.fori_loop(0, RPT // 16, comb, 0)
        pltpu.sync_copy(cout, out_hbm.at[pl.ds(s * RPT, RPT)])


def kernel(x, edge_index, W1_l, b1_l, W1_r, W2_l, b2_l, W2_r):
    src = jnp.concatenate(
        [edge_index[0], jnp.zeros((EP - E,), jnp.int32)])
    dst = jnp.concatenate(
        [edge_index[1], jnp.full((EP - E,), DISCARD, jnp.int32)])
    x0 = x[:, :HD]
    x1 = x[:, HD:]
    agg0, agg1, cnt = _sc_pass1(x0, x1, src, dst)

    W2 = jnp.concatenate([W2_l, W2_r], axis=0)
    tu = _tc_dense(cnt.reshape(NP, 1), x, agg0, agg1, W1_l,
                   b1_l.reshape(1, D), W1_r, W2)
    t = tu[:, 0]
    u = tu[:, 1]

    b2b = jnp.broadcast_to(b2_l, (16,))
    out = _sc_pass2(t, u, cnt, src, dst, b2b)
    return out[:N]
